# Initial kernel scaffold; baseline (speedup 1.0000x reference)
import functools

import jax
import jax.numpy as jnp
from jax import lax
from jax.experimental import pallas as pl
from jax.experimental.pallas import tpu as pltpu
from jax.experimental.pallas import tpu_sc as plsc

N = 10000
E = 320000
D_IN = 128
D_H = 128
D_OUT = 40
D3 = 64

NCORE = 2
NSUB = 16
NW = NCORE * NSUB
CHUNK = 128
CPW = 79
EPAD = NW * CPW * CHUNK
NPAD = 10112
RPS = NPAD // NSUB

_MESH = plsc.VectorSubcoreMesh(
    core_axis_name="c", subcore_axis_name="s",
    num_cores=NCORE, num_subcores=NSUB)


def _make_agg(D):
    @functools.partial(
        pl.kernel,
        out_type=jax.ShapeDtypeStruct((2 * NPAD, D), jnp.float32),
        mesh=_MESH,
        scratch_types=[
            pltpu.VMEM((CPW, CHUNK), jnp.int32),
            pltpu.VMEM((CPW, CHUNK), jnp.int32),
            pltpu.VMEM((CHUNK, D), jnp.float32),
            pltpu.VMEM_SHARED((NPAD, D), jnp.float32),
            pltpu.SemaphoreType.DMA,
        ],
    )
    def agg(h_hbm, src_hbm, dst_hbm, zeros_hbm, out_hbm,
            src_v, dst_v, rows_v, acc, sem):
        c = lax.axis_index("c")
        s = lax.axis_index("s")
        w = c * NSUB + s
        r0 = s * RPS
        pltpu.sync_copy(zeros_hbm.at[pl.ds(r0, RPS)], acc.at[pl.ds(r0, RPS)])
        pltpu.sync_copy(src_hbm.at[w], src_v)
        pltpu.sync_copy(dst_hbm.at[w], dst_v)
        plsc.subcore_barrier()

        def body(j, carry):
            pltpu.async_copy(h_hbm.at[src_v.at[j]], rows_v, sem).wait()
            pltpu.sync_copy(rows_v, acc.at[dst_v.at[j]], add=True)
            return carry

        lax.fori_loop(0, CPW, body, 0)
        plsc.subcore_barrier()
        pltpu.sync_copy(acc.at[pl.ds(r0, RPS)],
                        out_hbm.at[pl.ds(c * NPAD + r0, RPS)])

    return agg


_agg128 = _make_agg(D_H)
_agg64 = _make_agg(D3)


@functools.partial(
    pl.kernel,
    out_type=(jax.ShapeDtypeStruct((2 * NPAD, 16), jnp.float32),
              jax.ShapeDtypeStruct((2 * NPAD, 16), jnp.float32)),
    mesh=_MESH,
    scratch_types=[
        pltpu.VMEM((CPW, CHUNK), jnp.int32),
        pltpu.VMEM((CPW, CHUNK), jnp.int32),
        pltpu.VMEM((CHUNK, 16), jnp.float32),
        pltpu.VMEM_SHARED((NPAD, 16), jnp.float32),
        pltpu.VMEM_SHARED((NPAD, 16), jnp.float32),
    ],
)
def _deg_kernel(src_hbm, dst_hbm, zeros_hbm, ones_hbm,
                outdeg_hbm, indeg_hbm,
                src_v, dst_v, ones_v, acc_a, acc_b):
    c = lax.axis_index("c")
    s = lax.axis_index("s")
    w = c * NSUB + s
    r0 = s * RPS
    pltpu.sync_copy(zeros_hbm.at[pl.ds(r0, RPS)], acc_a.at[pl.ds(r0, RPS)])
    pltpu.sync_copy(zeros_hbm.at[pl.ds(r0, RPS)], acc_b.at[pl.ds(r0, RPS)])
    pltpu.sync_copy(ones_hbm, ones_v)
    pltpu.sync_copy(src_hbm.at[w], src_v)
    pltpu.sync_copy(dst_hbm.at[w], dst_v)
    plsc.subcore_barrier()

    def body(j, carry):
        pltpu.sync_copy(ones_v, acc_a.at[src_v.at[j]], add=True)
        pltpu.sync_copy(ones_v, acc_b.at[dst_v.at[j]], add=True)
        return carry

    lax.fori_loop(0, CPW, body, 0)
    plsc.subcore_barrier()
    pltpu.sync_copy(acc_a.at[pl.ds(r0, RPS)],
                    outdeg_hbm.at[pl.ds(c * NPAD + r0, RPS)])
    pltpu.sync_copy(acc_b.at[pl.ds(r0, RPS)],
                    indeg_hbm.at[pl.ds(c * NPAD + r0, RPS)])


_PREC = jax.lax.Precision.HIGHEST


def _tc_prep_body(degs_ref, degd_ref, x_ref, w1_ref, z_ref, ns_ref, nd_ref):
    out_deg = degs_ref[:NPAD, 0:1] + degs_ref[NPAD:, 0:1]
    in_deg = degd_ref[:NPAD, 0:1] + degd_ref[NPAD:, 0:1]
    ns = lax.rsqrt(jnp.maximum(out_deg, 1.0))
    nd = lax.rsqrt(jnp.maximum(in_deg, 1.0))
    ns_ref[...] = ns
    nd_ref[...] = nd
    z = jnp.dot(x_ref[...], w1_ref[...],
                preferred_element_type=jnp.float32, precision=_PREC)
    z_ref[...] = z * ns


def _tc_prep(degs, degd, x, w1):
    return pl.pallas_call(
        _tc_prep_body,
        out_shape=(jax.ShapeDtypeStruct((NPAD, D_H), jnp.float32),
                   jax.ShapeDtypeStruct((NPAD, 1), jnp.float32),
                   jax.ShapeDtypeStruct((NPAD, 1), jnp.float32)),
    )(degs, degd, x, w1)


def _tc_mid_body(agg_ref, nd_ref, b_ref, w_ref, ns_ref, z_ref):
    agg = agg_ref[:NPAD, :] + agg_ref[NPAD:, :]
    h = jnp.maximum(agg * nd_ref[...] + b_ref[...][None, :], 0.0)
    mask = lax.broadcasted_iota(jnp.int32, (NPAD, 1), 0) < N
    h = jnp.where(mask, h, 0.0)
    mu = jnp.sum(h) / (N * D_H)
    d = h - mu
    var = jnp.sum(jnp.where(mask, d * d, 0.0)) / (N * D_H)
    hn = jnp.where(mask, d * lax.rsqrt(var + 1e-5), 0.0)
    z = jnp.dot(hn, w_ref[...],
                preferred_element_type=jnp.float32, precision=_PREC)
    z_ref[...] = z * ns_ref[...]


def _tc_mid(agg, nd, b, w, ns, dout):
    return pl.pallas_call(
        _tc_mid_body,
        out_shape=jax.ShapeDtypeStruct((NPAD, dout), jnp.float32),
    )(agg, nd, b, w, ns)


def _tc_final_body(agg_ref, nd_ref, b_ref, out_ref):
    agg = agg_ref[:NPAD, :] + agg_ref[NPAD:, :]
    out_ref[...] = agg * nd_ref[...] + b_ref[...][None, :]


def _tc_final(agg, nd, b):
    return pl.pallas_call(
        _tc_final_body,
        out_shape=jax.ShapeDtypeStruct((NPAD, D3), jnp.float32),
    )(agg, nd, b)


def kernel(features, edge_index, W1, b1, W2, b2, W3, b3):
    src = edge_index[0]
    dst = edge_index[1]
    pad = jnp.full((EPAD - E,), N, dtype=jnp.int32)
    src3 = jnp.concatenate([src, pad]).reshape(NW, CPW, CHUNK)
    dst3 = jnp.concatenate([dst, pad]).reshape(NW, CPW, CHUNK)

    x = jnp.zeros((NPAD, D_IN), jnp.float32).at[:N].set(features)
    zeros128 = jnp.zeros((NPAD, D_H), jnp.float32)
    zeros64 = jnp.zeros((NPAD, D3), jnp.float32)
    zeros16 = jnp.zeros((NPAD, 16), jnp.float32)
    ones16 = jnp.ones((CHUNK, 16), jnp.float32)
    W3p = jnp.zeros((D_H, D3), jnp.float32).at[:, :D_OUT].set(W3)
    b3p = jnp.zeros((D3,), jnp.float32).at[:D_OUT].set(b3)

    degs, degd = _deg_kernel(src3, dst3, zeros16, ones16)
    z1, ns, nd = _tc_prep(degs, degd, x, W1)
    a1 = _agg128(z1, src3, dst3, zeros128)
    z2 = _tc_mid(a1, nd, b1, W2, ns)
    a2 = _agg128(z2, src3, dst3, zeros128)
    z3 = _tc_mid(a2, nd, b2, W3p, ns)
    a3 = _agg64(z3, src3, dst3, zeros64)
    outp = _tc_final(a3, nd, b3p)
    return outp[:N, :D_OUT]


# speedup vs baseline: 4.8109x; 4.8109x over previous
import functools

import jax
import jax.numpy as jnp
from jax import lax
from jax.experimental import pallas as pl
from jax.experimental.pallas import tpu as pltpu
from jax.experimental.pallas import tpu_sc as plsc

N = 10000
E = 320000
D_IN = 128
D_H = 128
D_OUT = 40
D3 = 64

NCORE = 2
NSUB = 16
NW = NCORE * NSUB
CHUNK = 128
CPW = 79
EPAD = NW * CPW * CHUNK
NPAD = 10112
RPS = NPAD // NSUB

_MESH = plsc.VectorSubcoreMesh(
    core_axis_name="c", subcore_axis_name="s",
    num_cores=NCORE, num_subcores=NSUB)


def _make_agg(D):
    @functools.partial(
        pl.kernel,
        out_type=jax.ShapeDtypeStruct((2 * NPAD, D), jnp.float32),
        mesh=_MESH,
        compiler_params=pltpu.CompilerParams(use_tc_tiling_on_sc=False),
        scratch_types=[
            pltpu.VMEM((CPW, CHUNK), jnp.int32),
            pltpu.VMEM((CPW, CHUNK), jnp.int32),
            pltpu.VMEM((CHUNK, D), jnp.float32),
            pltpu.VMEM_SHARED((NPAD, D), jnp.float32),
            pltpu.SemaphoreType.DMA,
        ],
    )
    def agg(h_hbm, src_hbm, dst_hbm, zeros_hbm, out_hbm,
            src_v, dst_v, rows_v, acc, sem):
        c = lax.axis_index("c")
        s = lax.axis_index("s")
        w = c * NSUB + s
        r0 = s * RPS
        pltpu.sync_copy(zeros_hbm.at[pl.ds(r0, RPS)], acc.at[pl.ds(r0, RPS)])
        pltpu.sync_copy(src_hbm.at[w], src_v)
        pltpu.sync_copy(dst_hbm.at[w], dst_v)
        plsc.subcore_barrier()

        def body(j, carry):
            pltpu.async_copy(h_hbm.at[src_v.at[j]], rows_v, sem).wait()
            pltpu.sync_copy(rows_v, acc.at[dst_v.at[j]], add=True)
            return carry

        lax.fori_loop(0, CPW, body, 0)
        plsc.subcore_barrier()
        pltpu.sync_copy(acc.at[pl.ds(r0, RPS)],
                        out_hbm.at[pl.ds(c * NPAD + r0, RPS)])

    return agg


_agg128 = _make_agg(D_H)
_agg64 = _make_agg(D3)


@functools.partial(
    pl.kernel,
    out_type=(jax.ShapeDtypeStruct((2 * NPAD, 16), jnp.float32),
              jax.ShapeDtypeStruct((2 * NPAD, 16), jnp.float32)),
    mesh=_MESH,
    compiler_params=pltpu.CompilerParams(use_tc_tiling_on_sc=False),
    scratch_types=[
        pltpu.VMEM((CPW, CHUNK), jnp.int32),
        pltpu.VMEM((CPW, CHUNK), jnp.int32),
        pltpu.VMEM((CHUNK, 16), jnp.float32),
        pltpu.VMEM_SHARED((NPAD, 16), jnp.float32),
        pltpu.VMEM_SHARED((NPAD, 16), jnp.float32),
    ],
)
def _deg_kernel(src_hbm, dst_hbm, zeros_hbm, ones_hbm,
                outdeg_hbm, indeg_hbm,
                src_v, dst_v, ones_v, acc_a, acc_b):
    c = lax.axis_index("c")
    s = lax.axis_index("s")
    w = c * NSUB + s
    r0 = s * RPS
    pltpu.sync_copy(zeros_hbm.at[pl.ds(r0, RPS)], acc_a.at[pl.ds(r0, RPS)])
    pltpu.sync_copy(zeros_hbm.at[pl.ds(r0, RPS)], acc_b.at[pl.ds(r0, RPS)])
    pltpu.sync_copy(ones_hbm, ones_v)
    pltpu.sync_copy(src_hbm.at[w], src_v)
    pltpu.sync_copy(dst_hbm.at[w], dst_v)
    plsc.subcore_barrier()

    def body(j, carry):
        pltpu.sync_copy(ones_v, acc_a.at[src_v.at[j]], add=True)
        pltpu.sync_copy(ones_v, acc_b.at[dst_v.at[j]], add=True)
        return carry

    lax.fori_loop(0, CPW, body, 0)
    plsc.subcore_barrier()
    pltpu.sync_copy(acc_a.at[pl.ds(r0, RPS)],
                    outdeg_hbm.at[pl.ds(c * NPAD + r0, RPS)])
    pltpu.sync_copy(acc_b.at[pl.ds(r0, RPS)],
                    indeg_hbm.at[pl.ds(c * NPAD + r0, RPS)])


_PREC = jax.lax.Precision.HIGHEST


def _tc_norms_body(degs_ref, degd_ref, ns_ref, nd_ref):
    out_deg = degs_ref[:NPAD, 0:1] + degs_ref[NPAD:, 0:1]
    in_deg = degd_ref[:NPAD, 0:1] + degd_ref[NPAD:, 0:1]
    ns_ref[...] = lax.rsqrt(jnp.maximum(out_deg, 1.0))
    nd_ref[...] = lax.rsqrt(jnp.maximum(in_deg, 1.0))


def _tc_norms(degs, degd):
    return pl.pallas_call(
        _tc_norms_body,
        out_shape=(jax.ShapeDtypeStruct((NPAD, 1), jnp.float32),
                   jax.ShapeDtypeStruct((NPAD, 1), jnp.float32)),
    )(degs, degd)


def _tc_z1_body(x_ref, w1_ref, ns_ref, z_ref):
    z = jnp.dot(x_ref[...], w1_ref[...],
                preferred_element_type=jnp.float32, precision=_PREC)
    z_ref[...] = z * ns_ref[...]


def _tc_z1(x, w1, ns):
    return pl.pallas_call(
        _tc_z1_body,
        out_shape=jax.ShapeDtypeStruct((NPAD, D_H), jnp.float32),
    )(x, w1, ns)


def _tc_mid_body(agg_ref, nd_ref, b_ref, w_ref, ns_ref, z_ref):
    agg = agg_ref[:NPAD, :] + agg_ref[NPAD:, :]
    h = jnp.maximum(agg * nd_ref[...] + b_ref[...][None, :], 0.0)
    mask = lax.broadcasted_iota(jnp.int32, (NPAD, 1), 0) < N
    h = jnp.where(mask, h, 0.0)
    mu = jnp.sum(h) / (N * D_H)
    d = h - mu
    var = jnp.sum(jnp.where(mask, d * d, 0.0)) / (N * D_H)
    hn = jnp.where(mask, d * lax.rsqrt(var + 1e-5), 0.0)
    z = jnp.dot(hn, w_ref[...],
                preferred_element_type=jnp.float32, precision=_PREC)
    z_ref[...] = z * ns_ref[...]


def _tc_mid(agg, nd, b, w, ns):
    return pl.pallas_call(
        _tc_mid_body,
        out_shape=jax.ShapeDtypeStruct((NPAD, w.shape[1]), jnp.float32),
    )(agg, nd, b, w, ns)


def _tc_final_body(agg_ref, nd_ref, b_ref, out_ref):
    agg = agg_ref[:NPAD, :] + agg_ref[NPAD:, :]
    out_ref[...] = agg * nd_ref[...] + b_ref[...][None, :]


def _tc_final(agg, nd, b):
    return pl.pallas_call(
        _tc_final_body,
        out_shape=jax.ShapeDtypeStruct((NPAD, D3), jnp.float32),
    )(agg, nd, b)


def kernel(features, edge_index, W1, b1, W2, b2, W3, b3):
    src = edge_index[0]
    dst = edge_index[1]
    pad = jnp.full((EPAD - E,), N, dtype=jnp.int32)
    src3 = jnp.concatenate([src, pad]).reshape(NW, CPW, CHUNK)
    dst3 = jnp.concatenate([dst, pad]).reshape(NW, CPW, CHUNK)

    x = jnp.zeros((NPAD, D_IN), jnp.float32).at[:N].set(features)
    zeros128 = jnp.zeros((NPAD, D_H), jnp.float32)
    zeros64 = jnp.zeros((NPAD, D3), jnp.float32)
    zeros16 = jnp.zeros((NPAD, 16), jnp.float32)
    ones16 = jnp.ones((CHUNK, 16), jnp.float32)
    W3p = jnp.zeros((D_H, D3), jnp.float32).at[:, :D_OUT].set(W3)
    b3p = jnp.zeros((D3,), jnp.float32).at[:D_OUT].set(b3)

    degs, degd = _deg_kernel(src3, dst3, zeros16, ones16)
    ns, nd = _tc_norms(degs, degd)
    z1 = _tc_z1(x, W1, ns)
    a1 = _agg128(z1, src3, dst3, zeros128)
    z2 = _tc_mid(a1, nd, b1, W2, ns)
    a2 = _agg128(z2, src3, dst3, zeros128)
    z3 = _tc_mid(a2, nd, b2, W3p, ns)
    a3 = _agg64(z3, src3, dst3, zeros64)
    outp = _tc_final(a3, nd, b3p)
    return outp[:N, :D_OUT]
